# baseline (speedup 1.0000x reference)
import functools

import jax
import jax.numpy as jnp
from jax import lax
from jax.experimental import pallas as pl
from jax.experimental.pallas import tpu as pltpu
from jax.experimental.pallas import tpu_sc as plsc

N_TOKENS = 32768
D_MODEL = 1024
N_EXP = 64
K = 2

ROUTE_BLK = 2048


def _routing_body(g_ref, ep_ref, rp_ref, loads_ref, run_ref, u_ref):
    i = pl.program_id(0)
    b = ROUTE_BLK

    @pl.when(i == 0)
    def _init():
        run_ref[...] = jnp.zeros_like(run_ref)
        ia = lax.broadcasted_iota(jnp.int32, (b, b), 0)
        ib = lax.broadcasted_iota(jnp.int32, (b, b), 1)
        u_ref[...] = (ia < ib).astype(jnp.bfloat16)

    gt = g_ref[...].T
    iota0 = lax.broadcasted_iota(jnp.int32, (N_EXP, b), 0)

    m1 = jnp.max(gt, axis=0, keepdims=True)
    e0 = jnp.min(jnp.where(gt == m1, iota0, N_EXP), axis=0, keepdims=True)
    oh0 = iota0 == e0
    g2 = jnp.where(oh0, -1e30, gt)
    m2 = jnp.max(g2, axis=0, keepdims=True)
    e1 = jnp.min(jnp.where(g2 == m2, iota0, N_EXP), axis=0, keepdims=True)
    oh1 = iota0 == e1

    cnt = (oh0 | oh1).astype(jnp.bfloat16)
    excl = lax.dot_general(cnt, u_ref[...], (((1,), (0,)), ((), ())),
                           preferred_element_type=jnp.float32)
    excl = excl + run_ref[...]

    ei = excl.astype(jnp.int32)
    rp_ref[...] = jnp.sum(
        jnp.where(oh0, ei, 0) + jnp.where(oh1, ei << 15, 0), axis=0)

    new_run = run_ref[...] + jnp.sum(cnt.astype(jnp.float32), axis=1,
                                     keepdims=True)
    run_ref[...] = new_run

    ep_ref[...] = e0[0] | (e1[0] << 8)
    loads_ref[...] = new_run[:, 0].astype(jnp.int32)


def _routing(gates, interpret=False):
    n = gates.shape[0]
    grid = n // ROUTE_BLK
    out_shapes = (
        jax.ShapeDtypeStruct((n,), jnp.int32),
        jax.ShapeDtypeStruct((n,), jnp.int32),
        jax.ShapeDtypeStruct((N_EXP,), jnp.int32),
    )
    tok_spec = pl.BlockSpec((ROUTE_BLK,), lambda i: (i,))
    return pl.pallas_call(
        _routing_body,
        grid=(grid,),
        in_specs=[pl.BlockSpec((ROUTE_BLK, N_EXP), lambda i: (i, 0))],
        out_specs=(tok_spec, tok_spec,
                   pl.BlockSpec((N_EXP,), lambda i: (0,))),
        out_shape=out_shapes,
        scratch_shapes=[pltpu.VMEM((N_EXP, 1), jnp.float32),
                        pltpu.VMEM((ROUTE_BLK, ROUTE_BLK), jnp.bfloat16)],
        interpret=interpret,
    )(gates)


def _make_dispatch():
    info = plsc.get_sparse_core_info()
    nw = info.num_cores * info.num_subcores
    tpw = N_TOKENS // nw
    C = 32
    NBUF = 3
    nch = tpw // C
    nrounds = nch // NBUF
    mesh = plsc.VectorSubcoreMesh(core_axis_name="c", subcore_axis_name="s")

    @functools.partial(
        pl.kernel,
        out_type=jax.ShapeDtypeStruct((N_TOKENS * K, D_MODEL), jnp.float32),
        mesh=mesh,
        compiler_params=pltpu.CompilerParams(needs_layout_passes=False),
        scratch_types=[
            pltpu.VMEM((NBUF, C, D_MODEL), jnp.float32),
            pltpu.VMEM((tpw,), jnp.int32),
            pltpu.VMEM((tpw,), jnp.int32),
            pltpu.VMEM((N_EXP,), jnp.int32),
            pltpu.VMEM((N_EXP,), jnp.int32),
            pltpu.VMEM((2, tpw // C, C), jnp.int32),
            pltpu.SemaphoreType.DMA,
            [pltpu.SemaphoreType.DMA] * NBUF,
            [pltpu.SemaphoreType.DMA] * NBUF,
        ],
    )
    def dispatch(in_hbm, ep_hbm, rp_hbm, loads_hbm, out_hbm,
                 buf, ep_v, rp_v, loads_v, cum_v, dst_tbl,
                 msem, lsems, osems):
        wid = lax.axis_index("s") * info.num_cores + lax.axis_index("c")
        base = wid * tpw

        pltpu.async_copy(loads_hbm, loads_v, msem).wait()
        pltpu.async_copy(ep_hbm.at[pl.ds(base, tpw)], ep_v, msem).wait()
        pltpu.async_copy(rp_hbm.at[pl.ds(base, tpw)], rp_v, msem).wait()

        carry = jnp.float32(0)
        for j in range(N_EXP // 16):
            x = loads_v[pl.ds(j * 16, 16)].astype(jnp.float32)
            inc = plsc.cumsum(x)
            cum_v[pl.ds(j * 16, 16)] = (inc - x + carry).astype(jnp.int32)
            carry = carry + jnp.sum(x)

        for k in range(nch):
            for j in range(C // 16):
                sl = pl.ds(k * C + j * 16, 16)
                jj = pl.ds(j * 16, 16)
                ep = ep_v[sl]
                rp = rp_v[sl]
                dst_tbl[0, k, jj] = (
                    plsc.load_gather(cum_v, [ep & 0xFF]) + (rp & 0x7FFF))
                dst_tbl[1, k, jj] = (
                    plsc.load_gather(cum_v, [ep >> 8]) + (rp >> 15))

        def wait_scatter(p):
            for _ in range(2):
                pltpu.make_async_copy(
                    buf.at[p], out_hbm.at[dst_tbl.at[0, 0]], osems[p]).wait()

        def load(k, p):
            pltpu.async_copy(
                in_hbm.at[pl.ds(base + k * C, C)], buf.at[p], lsems[p])

        def process(k, p):
            pltpu.make_async_copy(
                in_hbm.at[pl.ds(base, C)], buf.at[p], lsems[p]).wait()
            pltpu.async_copy(buf.at[p], out_hbm.at[dst_tbl.at[0, k]],
                             osems[p])
            pltpu.async_copy(buf.at[p], out_hbm.at[dst_tbl.at[1, k]],
                             osems[p])

        def body(r, _):
            for p in range(NBUF):
                k = r * NBUF + p

                @pl.when(r > 0)
                def _():
                    wait_scatter(p)

                load(k, p)
            for p in range(NBUF):
                process(r * NBUF + p, p)
            return 0

        lax.fori_loop(0, nrounds, body, 0)
        for t, p in enumerate(range(nch - nrounds * NBUF)):
            k = nrounds * NBUF + t
            wait_scatter(p)
            load(k, p)
        for t, p in enumerate(range(nch - nrounds * NBUF)):
            process(nrounds * NBUF + t, p)
        for p in range(NBUF):
            wait_scatter(p)

    return dispatch


_dispatch = None


def kernel(in_flow, gates):
    global _dispatch
    if _dispatch is None:
        _dispatch = _make_dispatch()
    ep, rp, loads = _routing(gates)
    out = _dispatch(in_flow, ep, rp, loads)
    return out, loads


# speedup vs baseline: 1.0734x; 1.0734x over previous
import functools

import jax
import jax.numpy as jnp
from jax import lax
from jax.experimental import pallas as pl
from jax.experimental.pallas import tpu as pltpu
from jax.experimental.pallas import tpu_sc as plsc

N_TOKENS = 32768
D_MODEL = 1024
N_EXP = 64
K = 2

ROUTE_BLK = 2048


def _routing_body(g_ref, ep_ref, rp_ref, loads_ref, run_ref, u_ref):
    i = pl.program_id(0)
    b = ROUTE_BLK

    @pl.when(i == 0)
    def _init():
        run_ref[...] = jnp.zeros_like(run_ref)
        ia = lax.broadcasted_iota(jnp.int32, (b, b), 0)
        ib = lax.broadcasted_iota(jnp.int32, (b, b), 1)
        u_ref[...] = (ia < ib).astype(jnp.bfloat16)

    gt = g_ref[...]
    iota0 = lax.broadcasted_iota(jnp.int32, (N_EXP, b), 0)

    m1 = jnp.max(gt, axis=0, keepdims=True)
    e0 = jnp.min(jnp.where(gt == m1, iota0, N_EXP), axis=0, keepdims=True)
    oh0 = iota0 == e0
    g2 = jnp.where(oh0, -1e30, gt)
    m2 = jnp.max(g2, axis=0, keepdims=True)
    e1 = jnp.min(jnp.where(g2 == m2, iota0, N_EXP), axis=0, keepdims=True)
    oh1 = iota0 == e1

    cnt = (oh0 | oh1).astype(jnp.bfloat16)
    excl = lax.dot_general(cnt, u_ref[...], (((1,), (0,)), ((), ())),
                           preferred_element_type=jnp.float32)
    excl = excl + run_ref[...]

    ei = excl.astype(jnp.int32)
    rp_ref[...] = jnp.sum(
        jnp.where(oh0, ei, 0) + jnp.where(oh1, ei << 15, 0), axis=0)

    new_run = run_ref[...] + jnp.sum(cnt.astype(jnp.float32), axis=1,
                                     keepdims=True)
    run_ref[...] = new_run

    ep_ref[...] = e0[0] | (e1[0] << 8)
    loads_ref[...] = new_run[:, 0].astype(jnp.int32)


def _routing(gates, interpret=False):
    n = gates.shape[0]
    gt = gates.T
    grid = n // ROUTE_BLK
    out_shapes = (
        jax.ShapeDtypeStruct((n,), jnp.int32),
        jax.ShapeDtypeStruct((n,), jnp.int32),
        jax.ShapeDtypeStruct((N_EXP,), jnp.int32),
    )
    tok_spec = pl.BlockSpec((ROUTE_BLK,), lambda i: (i,))
    return pl.pallas_call(
        _routing_body,
        grid=(grid,),
        in_specs=[pl.BlockSpec((N_EXP, ROUTE_BLK), lambda i: (0, i))],
        out_specs=(tok_spec, tok_spec,
                   pl.BlockSpec((N_EXP,), lambda i: (0,))),
        out_shape=out_shapes,
        scratch_shapes=[pltpu.VMEM((N_EXP, 1), jnp.float32),
                        pltpu.VMEM((ROUTE_BLK, ROUTE_BLK), jnp.bfloat16)],
        interpret=interpret,
    )(gt)


def _make_dispatch():
    info = plsc.get_sparse_core_info()
    nw = info.num_cores * info.num_subcores
    tpw = N_TOKENS // nw
    C = 32
    NBUF = 3
    nch = tpw // C
    nrounds = nch // NBUF
    mesh = plsc.VectorSubcoreMesh(core_axis_name="c", subcore_axis_name="s")

    @functools.partial(
        pl.kernel,
        out_type=jax.ShapeDtypeStruct((N_TOKENS * K, D_MODEL), jnp.float32),
        mesh=mesh,
        compiler_params=pltpu.CompilerParams(needs_layout_passes=False),
        scratch_types=[
            pltpu.VMEM((NBUF, C, D_MODEL), jnp.float32),
            pltpu.VMEM((tpw,), jnp.int32),
            pltpu.VMEM((tpw,), jnp.int32),
            pltpu.VMEM((N_EXP,), jnp.int32),
            pltpu.VMEM((N_EXP,), jnp.int32),
            pltpu.VMEM((2, tpw // C, C), jnp.int32),
            pltpu.SemaphoreType.DMA,
            [pltpu.SemaphoreType.DMA] * NBUF,
            [pltpu.SemaphoreType.DMA] * NBUF,
        ],
    )
    def dispatch(in_hbm, ep_hbm, rp_hbm, loads_hbm, out_hbm,
                 buf, ep_v, rp_v, loads_v, cum_v, dst_tbl,
                 msem, lsems, osems):
        wid = lax.axis_index("s") * info.num_cores + lax.axis_index("c")
        base = wid * tpw

        pltpu.async_copy(loads_hbm, loads_v, msem).wait()
        pltpu.async_copy(ep_hbm.at[pl.ds(base, tpw)], ep_v, msem).wait()
        pltpu.async_copy(rp_hbm.at[pl.ds(base, tpw)], rp_v, msem).wait()

        carry = jnp.float32(0)
        for j in range(N_EXP // 16):
            x = loads_v[pl.ds(j * 16, 16)].astype(jnp.float32)
            inc = plsc.cumsum(x)
            cum_v[pl.ds(j * 16, 16)] = (inc - x + carry).astype(jnp.int32)
            carry = carry + jnp.sum(x)

        for k in range(nch):
            for j in range(C // 16):
                sl = pl.ds(k * C + j * 16, 16)
                jj = pl.ds(j * 16, 16)
                ep = ep_v[sl]
                rp = rp_v[sl]
                dst_tbl[0, k, jj] = (
                    plsc.load_gather(cum_v, [ep & 0xFF]) + (rp & 0x7FFF))
                dst_tbl[1, k, jj] = (
                    plsc.load_gather(cum_v, [ep >> 8]) + (rp >> 15))

        def wait_scatter(p):
            for _ in range(2):
                pltpu.make_async_copy(
                    buf.at[p], out_hbm.at[dst_tbl.at[0, 0]], osems[p]).wait()

        def load(k, p):
            pltpu.async_copy(
                in_hbm.at[pl.ds(base + k * C, C)], buf.at[p], lsems[p])

        def process(k, p):
            pltpu.make_async_copy(
                in_hbm.at[pl.ds(base, C)], buf.at[p], lsems[p]).wait()
            pltpu.async_copy(buf.at[p], out_hbm.at[dst_tbl.at[0, k]],
                             osems[p])
            pltpu.async_copy(buf.at[p], out_hbm.at[dst_tbl.at[1, k]],
                             osems[p])

        def body(r, _):
            for p in range(NBUF):
                k = r * NBUF + p

                @pl.when(r > 0)
                def _():
                    wait_scatter(p)

                load(k, p)
            for p in range(NBUF):
                process(r * NBUF + p, p)
            return 0

        lax.fori_loop(0, nrounds, body, 0)
        for t, p in enumerate(range(nch - nrounds * NBUF)):
            k = nrounds * NBUF + t
            wait_scatter(p)
            load(k, p)
        for t, p in enumerate(range(nch - nrounds * NBUF)):
            process(nrounds * NBUF + t, p)
        for p in range(NBUF):
            wait_scatter(p)

    return dispatch


_dispatch = None


def kernel(in_flow, gates):
    global _dispatch
    if _dispatch is None:
        _dispatch = _make_dispatch()
    ep, rp, loads = _routing(gates)
    out = _dispatch(in_flow, ep, rp, loads)
    return out, loads


# speedup vs baseline: 1.1046x; 1.0291x over previous
import functools

import jax
import jax.numpy as jnp
from jax import lax
from jax.experimental import pallas as pl
from jax.experimental.pallas import tpu as pltpu
from jax.experimental.pallas import tpu_sc as plsc

N_TOKENS = 32768
D_MODEL = 1024
N_EXP = 64
K = 2

ROUTE_BLK = 2048


def _routing_body(g_ref, ep_ref, rp_ref, loads_ref, run_ref, u_ref):
    i = pl.program_id(0)
    b = ROUTE_BLK

    @pl.when(i == 0)
    def _init():
        run_ref[...] = jnp.zeros_like(run_ref)
        ia = lax.broadcasted_iota(jnp.int32, (b, b), 0)
        ib = lax.broadcasted_iota(jnp.int32, (b, b), 1)
        u_ref[...] = (ia < ib).astype(jnp.bfloat16)

    gt = g_ref[...]
    iota0 = lax.broadcasted_iota(jnp.int32, (N_EXP, b), 0)

    m1 = jnp.max(gt, axis=0, keepdims=True)
    e0 = jnp.min(jnp.where(gt == m1, iota0, N_EXP), axis=0, keepdims=True)
    oh0 = iota0 == e0
    g2 = jnp.where(oh0, -1e30, gt)
    m2 = jnp.max(g2, axis=0, keepdims=True)
    e1 = jnp.min(jnp.where(g2 == m2, iota0, N_EXP), axis=0, keepdims=True)
    oh1 = iota0 == e1

    cnt = (oh0 | oh1).astype(jnp.bfloat16)
    excl = lax.dot_general(cnt, u_ref[...], (((1,), (0,)), ((), ())),
                           preferred_element_type=jnp.float32)
    excl = excl + run_ref[...]

    ei = excl.astype(jnp.int32)
    rp_ref[...] = jnp.sum(
        jnp.where(oh0, ei, 0) + jnp.where(oh1, ei << 15, 0), axis=0)

    new_run = run_ref[...] + jnp.sum(cnt.astype(jnp.float32), axis=1,
                                     keepdims=True)
    run_ref[...] = new_run

    ep_ref[...] = e0[0] | (e1[0] << 8)
    loads_ref[...] = new_run[:, 0].astype(jnp.int32)


def _routing(gates, interpret=False):
    n = gates.shape[0]
    gt = gates.T
    grid = n // ROUTE_BLK
    out_shapes = (
        jax.ShapeDtypeStruct((n,), jnp.int32),
        jax.ShapeDtypeStruct((n,), jnp.int32),
        jax.ShapeDtypeStruct((N_EXP,), jnp.int32),
    )
    tok_spec = pl.BlockSpec((ROUTE_BLK,), lambda i: (i,))
    return pl.pallas_call(
        _routing_body,
        grid=(grid,),
        in_specs=[pl.BlockSpec((N_EXP, ROUTE_BLK), lambda i: (0, i))],
        out_specs=(tok_spec, tok_spec,
                   pl.BlockSpec((N_EXP,), lambda i: (0,))),
        out_shape=out_shapes,
        scratch_shapes=[pltpu.VMEM((N_EXP, 1), jnp.float32),
                        pltpu.VMEM((ROUTE_BLK, ROUTE_BLK), jnp.bfloat16)],
        interpret=interpret,
    )(gt)


def _make_dispatch():
    info = plsc.get_sparse_core_info()
    nw = info.num_cores * info.num_subcores
    tpw = N_TOKENS // nw
    C = 32
    NBUF = 3
    nch = tpw // C
    nrounds = nch // NBUF
    mesh = plsc.VectorSubcoreMesh(core_axis_name="c", subcore_axis_name="s")

    @functools.partial(
        pl.kernel,
        out_type=jax.ShapeDtypeStruct((N_TOKENS * K, D_MODEL), jnp.float32),
        mesh=mesh,
        compiler_params=pltpu.CompilerParams(needs_layout_passes=False),
        scratch_types=[
            pltpu.VMEM((NBUF, C, D_MODEL), jnp.float32),
            pltpu.VMEM((tpw,), jnp.int32),
            pltpu.VMEM((tpw,), jnp.int32),
            pltpu.VMEM((N_EXP,), jnp.int32),
            pltpu.VMEM((N_EXP,), jnp.int32),
            pltpu.VMEM((2, tpw // C, C), jnp.int32),
            pltpu.SemaphoreType.DMA,
            [pltpu.SemaphoreType.DMA] * NBUF,
            [pltpu.SemaphoreType.DMA] * NBUF,
        ],
    )
    def dispatch(in_hbm, ep_hbm, rp_hbm, loads_hbm, out_hbm,
                 buf, ep_v, rp_v, loads_v, cum_v, dst_tbl,
                 msem, lsems, osems):
        wid = lax.axis_index("s") * info.num_cores + lax.axis_index("c")
        base = wid * tpw

        def load(k, p):
            pltpu.async_copy(
                in_hbm.at[pl.ds(base + k * C, C)], buf.at[p], lsems[p])

        for p in range(NBUF):
            load(p, p)

        c0 = pltpu.async_copy(loads_hbm, loads_v, msem)
        c1 = pltpu.async_copy(ep_hbm.at[pl.ds(base, tpw)], ep_v, msem)
        c2 = pltpu.async_copy(rp_hbm.at[pl.ds(base, tpw)], rp_v, msem)
        c0.wait()
        c1.wait()
        c2.wait()

        carry = jnp.float32(0)
        for j in range(N_EXP // 16):
            x = loads_v[pl.ds(j * 16, 16)].astype(jnp.float32)
            inc = plsc.cumsum(x)
            cum_v[pl.ds(j * 16, 16)] = (inc - x + carry).astype(jnp.int32)
            carry = carry + jnp.sum(x)

        for k in range(nch):
            for j in range(C // 16):
                sl = pl.ds(k * C + j * 16, 16)
                jj = pl.ds(j * 16, 16)
                ep = ep_v[sl]
                rp = rp_v[sl]
                dst_tbl[0, k, jj] = (
                    plsc.load_gather(cum_v, [ep & 0xFF]) + (rp & 0x7FFF))
                dst_tbl[1, k, jj] = (
                    plsc.load_gather(cum_v, [ep >> 8]) + (rp >> 15))

        def wait_scatter(p):
            for _ in range(2):
                pltpu.make_async_copy(
                    buf.at[p], out_hbm.at[dst_tbl.at[0, 0]], osems[p]).wait()

        def process(k, p):
            pltpu.make_async_copy(
                in_hbm.at[pl.ds(base, C)], buf.at[p], lsems[p]).wait()
            pltpu.async_copy(buf.at[p], out_hbm.at[dst_tbl.at[0, k]],
                             osems[p])
            pltpu.async_copy(buf.at[p], out_hbm.at[dst_tbl.at[1, k]],
                             osems[p])

        def body(r, _):
            for p in range(NBUF):
                k = r * NBUF + p

                @pl.when(r > 0)
                def _():
                    wait_scatter(p)
                    load(k, p)

            for p in range(NBUF):
                process(r * NBUF + p, p)
            return 0

        lax.fori_loop(0, nrounds, body, 0)
        for t, p in enumerate(range(nch - nrounds * NBUF)):
            k = nrounds * NBUF + t
            wait_scatter(p)
            load(k, p)
        for t, p in enumerate(range(nch - nrounds * NBUF)):
            process(nrounds * NBUF + t, p)
        for p in range(NBUF):
            wait_scatter(p)

    return dispatch


_dispatch = None


def kernel(in_flow, gates):
    global _dispatch
    if _dispatch is None:
        _dispatch = _make_dispatch()
    ep, rp, loads = _routing(gates)
    out = _dispatch(in_flow, ep, rp, loads)
    return out, loads
